# baseline (speedup 1.0000x reference)
import functools

import jax
import jax.numpy as jnp
from jax.experimental import pallas as pl
from jax.experimental.pallas import tpu as pltpu

_H = 12
_HALO = 10
_OFFS = (-3, -2, -1, 1, 2, 3, -10, -5, 5, 10)
_GROUP = (0, 0, 0, 0, 0, 0, 1, 1, 1, 1, 2, 2)


def _fused_kernel(xp_ref, wqT_ref, wkvT_ref, bq_ref, bkv_ref, woutT_ref,
                  bout_ref, lw_ref, out_ref, *, blk, S, D, dh):
    i = pl.program_id(1)
    W = blk + 2 * _HALO
    f32 = jnp.float32

    xh = xp_ref[0, pl.ds(i * blk, W), :]
    x0 = xp_ref[0, pl.ds(_HALO, 1), :]
    xS = xp_ref[0, pl.ds(S + _HALO - 1, 1), :]
    xkv = jnp.concatenate([xh, x0, xS], axis=0)

    q = (jnp.dot(xh[_HALO:_HALO + blk], wqT_ref[...],
                 preferred_element_type=f32).astype(f32) + bq_ref[...])
    kv = (jnp.dot(xkv, wkvT_ref[...], preferred_element_type=f32)
          .astype(f32) + bkv_ref[...])
    kh, vh = kv[:, :D], kv[:, D:]

    scale = dh ** -0.5
    rows = jax.lax.broadcasted_iota(jnp.int32, (D, _H), 0)
    cols = jax.lax.broadcasted_iota(jnp.int32, (D, _H), 1)
    Ms = jnp.where(rows // dh == cols, scale, 0.0).astype(f32)
    rows_e = jax.lax.broadcasted_iota(jnp.int32, (_H, D), 0)
    cols_e = jax.lax.broadcasted_iota(jnp.int32, (_H, D), 1)
    E = jnp.where(cols_e // dh == rows_e, 1.0, 0.0).astype(f32)

    ks = [kh[_HALO + o:_HALO + o + blk] for o in _OFFS]
    ks += [kh[W:W + 1], kh[W + 1:W + 2]]
    vs = [vh[_HALO + o:_HALO + o + blk] for o in _OFFS]
    vs += [vh[W:W + 1], vh[W + 1:W + 2]]

    Ls = [jnp.dot(q * ks[a], Ms, preferred_element_type=f32) + lw_ref[a]
          for a in range(12)]
    m = functools.reduce(jnp.maximum, Ls)
    acc = jnp.zeros((blk, D), f32)
    Z = jnp.zeros((blk, _H), f32)
    for a in range(12):
        e = jnp.exp(Ls[a] - m)
        Z = Z + e
        acc = acc + jnp.dot(e, E, preferred_element_type=f32) * vs[a]
    attn_out = acc / jnp.dot(Z, E, preferred_element_type=f32)

    out_ref[0] = (jnp.dot(attn_out.astype(jnp.bfloat16), woutT_ref[...],
                          preferred_element_type=f32) + bout_ref[...])


def kernel(x, Wqkv, bqkv, Wout, bout, group_scale, anchor_idx):
    B, S, D = x.shape
    dh = D // _H
    blk = 512
    nb = S // blk

    x_pad = jnp.concatenate([
        jnp.broadcast_to(x[:, :1], (B, _HALO, D)), x,
        jnp.broadcast_to(x[:, -1:], (B, _HALO, D))], axis=1).astype(jnp.bfloat16)
    WqT = Wqkv[:D].T.astype(jnp.bfloat16)
    WkvT = Wqkv[D:].T.astype(jnp.bfloat16)
    bq = bqkv[:D].reshape(1, D)
    bkv = bqkv[D:].reshape(1, 2 * D)
    WoutT = Wout.T.astype(jnp.bfloat16)
    bout2 = bout.reshape(1, D)
    lw = jnp.log(jax.nn.softmax(group_scale))[jnp.array(_GROUP)]

    grid = (B, nb)
    return pl.pallas_call(
        functools.partial(_fused_kernel, blk=blk, S=S, D=D, dh=dh),
        grid=grid,
        in_specs=[
            pl.BlockSpec((1, S + 2 * _HALO, D), lambda b, i: (b, 0, 0)),
            pl.BlockSpec((D, D), lambda b, i: (0, 0)),
            pl.BlockSpec((D, 2 * D), lambda b, i: (0, 0)),
            pl.BlockSpec((1, D), lambda b, i: (0, 0)),
            pl.BlockSpec((1, 2 * D), lambda b, i: (0, 0)),
            pl.BlockSpec((D, D), lambda b, i: (0, 0)),
            pl.BlockSpec((1, D), lambda b, i: (0, 0)),
            pl.BlockSpec(memory_space=pltpu.SMEM),
        ],
        out_specs=pl.BlockSpec((1, blk, D), lambda b, i: (b, i, 0)),
        out_shape=jax.ShapeDtypeStruct((B, S, D), jnp.float32),
    )(x_pad, WqT, WkvT, bq, bkv, WoutT, bout2, lw)


# speedup vs baseline: 1.0158x; 1.0158x over previous
import functools

import jax
import jax.numpy as jnp
from jax.experimental import pallas as pl
from jax.experimental.pallas import tpu as pltpu

_H = 12
_HALO = 10
_OFFS = (-3, -2, -1, 1, 2, 3, -10, -5, 5, 10)
_GROUP = (0, 0, 0, 0, 0, 0, 1, 1, 1, 1, 2, 2)

_DNT = (((1,), (1,)), ((), ()))


def _fused_kernel(xp_ref, wqkv_ref, bq_ref, bkv_ref, wout_ref,
                  bout_ref, lw_ref, out_ref, *, blk, S, D, dh):
    i = pl.program_id(1)
    W = blk + 2 * _HALO
    f32 = jnp.float32

    xh = xp_ref[0, pl.ds(i * blk, W), :]
    x0 = xp_ref[0, pl.ds(_HALO, 1), :]
    xS = xp_ref[0, pl.ds(S + _HALO - 1, 1), :]
    xkv = jnp.concatenate([xh, x0, xS], axis=0)

    q = (jax.lax.dot_general(xh[_HALO:_HALO + blk], wqkv_ref[0:D], _DNT,
                             preferred_element_type=f32).astype(f32)
         + bq_ref[...])
    kv = (jax.lax.dot_general(xkv, wqkv_ref[D:3 * D], _DNT,
                              preferred_element_type=f32).astype(f32)
          + bkv_ref[...])
    kh, vh = kv[:, :D], kv[:, D:]

    scale = dh ** -0.5
    rows = jax.lax.broadcasted_iota(jnp.int32, (D, _H), 0)
    cols = jax.lax.broadcasted_iota(jnp.int32, (D, _H), 1)
    Ms = jnp.where(rows // dh == cols, scale, 0.0).astype(f32)
    rows_e = jax.lax.broadcasted_iota(jnp.int32, (_H, D), 0)
    cols_e = jax.lax.broadcasted_iota(jnp.int32, (_H, D), 1)
    E = jnp.where(cols_e // dh == rows_e, 1.0, 0.0).astype(f32)

    ks = [kh[_HALO + o:_HALO + o + blk] for o in _OFFS]
    ks += [kh[W:W + 1], kh[W + 1:W + 2]]
    vs = [vh[_HALO + o:_HALO + o + blk] for o in _OFFS]
    vs += [vh[W:W + 1], vh[W + 1:W + 2]]

    Ls = [jnp.dot(q * ks[a], Ms, preferred_element_type=f32) + lw_ref[a]
          for a in range(12)]
    m = functools.reduce(jnp.maximum, Ls)
    acc = jnp.zeros((blk, D), f32)
    Z = jnp.zeros((blk, _H), f32)
    for a in range(12):
        e = jnp.exp(Ls[a] - m)
        Z = Z + e
        acc = acc + jnp.dot(e, E, preferred_element_type=f32) * vs[a]
    attn_out = acc / jnp.dot(Z, E, preferred_element_type=f32)

    out_ref[0] = (jax.lax.dot_general(attn_out.astype(jnp.bfloat16),
                                      wout_ref[...], _DNT,
                                      preferred_element_type=f32)
                  + bout_ref[...])


def kernel(x, Wqkv, bqkv, Wout, bout, group_scale, anchor_idx):
    B, S, D = x.shape
    dh = D // _H
    blk = 512
    nb = S // blk

    x_pad = jnp.concatenate([
        jnp.broadcast_to(x[:, :1], (B, _HALO, D)), x,
        jnp.broadcast_to(x[:, -1:], (B, _HALO, D))],
        axis=1).astype(jnp.bfloat16)
    Wqkv_b = Wqkv.astype(jnp.bfloat16)
    Wout_b = Wout.astype(jnp.bfloat16)
    bq = bqkv[:D].reshape(1, D)
    bkv = bqkv[D:].reshape(1, 2 * D)
    bout2 = bout.reshape(1, D)
    lw = jnp.log(jax.nn.softmax(group_scale))[jnp.array(_GROUP)]

    grid = (B, nb)
    return pl.pallas_call(
        functools.partial(_fused_kernel, blk=blk, S=S, D=D, dh=dh),
        grid=grid,
        in_specs=[
            pl.BlockSpec((1, S + 2 * _HALO, D), lambda b, i: (b, 0, 0)),
            pl.BlockSpec((3 * D, D), lambda b, i: (0, 0)),
            pl.BlockSpec((1, D), lambda b, i: (0, 0)),
            pl.BlockSpec((1, 2 * D), lambda b, i: (0, 0)),
            pl.BlockSpec((D, D), lambda b, i: (0, 0)),
            pl.BlockSpec((1, D), lambda b, i: (0, 0)),
            pl.BlockSpec(memory_space=pltpu.SMEM),
        ],
        out_specs=pl.BlockSpec((1, blk, D), lambda b, i: (b, i, 0)),
        out_shape=jax.ShapeDtypeStruct((B, S, D), jnp.float32),
        compiler_params=pltpu.CompilerParams(
            dimension_semantics=("parallel", "parallel")),
    )(x_pad, Wqkv_b, bq, bkv, Wout_b, bout2, lw)


# speedup vs baseline: 1.4146x; 1.3927x over previous
import functools

import jax
import jax.numpy as jnp
from jax.experimental import pallas as pl
from jax.experimental.pallas import tpu as pltpu

_H = 12
_HALO = 10
_OFFS = (-3, -2, -1, 1, 2, 3, -10, -5, 5, 10)
_GROUP = (0, 0, 0, 0, 0, 0, 1, 1, 1, 1, 2, 2)

_DNT = (((1,), (1,)), ((), ()))


def _fused_kernel(x_ref, wqkv_ref, bq_ref, bkv_ref, wout_ref,
                  bout_ref, lw_ref, out_ref, *, blk, S, D, dh, nb):
    i = pl.program_id(1)
    W = blk + 2 * _HALO
    f32 = jnp.float32
    bf16 = jnp.bfloat16

    W2 = blk + 32
    wc = pl.multiple_of(jnp.clip(i * blk - 16, 0, S - W2), 16)
    xw = x_ref[0, pl.ds(wc, W2), :].astype(bf16)
    xh0 = jnp.concatenate(
        [jnp.broadcast_to(xw[0:1], (_HALO, D)), xw[:W - _HALO]], axis=0)
    xhN = jnp.concatenate(
        [xw[22:], jnp.broadcast_to(xw[W2 - 1:W2], (_HALO, D))], axis=0)
    xh = jnp.where(i == 0, xh0, jnp.where(i == nb - 1, xhN, xw[6:6 + W]))
    x0 = x_ref[0, pl.ds(0, 1), :].astype(bf16)
    xS = x_ref[0, pl.ds(S - 1, 1), :].astype(bf16)
    xkv = jnp.concatenate([xh, x0, xS], axis=0)

    q = (jax.lax.dot_general(xh[_HALO:_HALO + blk], wqkv_ref[0:D], _DNT,
                             preferred_element_type=f32)
         + bq_ref[...])
    kv = (jax.lax.dot_general(xkv, wqkv_ref[D:3 * D], _DNT,
                              preferred_element_type=f32)
          + bkv_ref[...])
    kh, vh = kv[:, :D], kv[:, D:]

    scale = dh ** -0.5
    rows = jax.lax.broadcasted_iota(jnp.int32, (D, _H), 0)
    cols = jax.lax.broadcasted_iota(jnp.int32, (D, _H), 1)
    Ms = jnp.where(rows // dh == cols, scale, 0.0).astype(f32)
    rows_e = jax.lax.broadcasted_iota(jnp.int32, (_H, D), 0)
    cols_e = jax.lax.broadcasted_iota(jnp.int32, (_H, D), 1)
    E = jnp.where(cols_e // dh == rows_e, 1.0, 0.0).astype(f32)

    ks = [kh[_HALO + o:_HALO + o + blk] for o in _OFFS]
    ks += [kh[W:W + 1], kh[W + 1:W + 2]]
    vs = [vh[_HALO + o:_HALO + o + blk] for o in _OFFS]
    vs += [vh[W:W + 1], vh[W + 1:W + 2]]

    Ls = [jnp.dot(q * ks[a], Ms, preferred_element_type=f32) + lw_ref[a]
          for a in range(12)]
    m = functools.reduce(jnp.maximum, Ls)
    acc = jnp.zeros((blk, D), f32)
    Z = jnp.zeros((blk, _H), f32)
    for a in range(12):
        e = jnp.exp(Ls[a] - m)
        Z = Z + e
        acc = acc + jnp.dot(e, E, preferred_element_type=f32) * vs[a]
    attn_out = acc / jnp.dot(Z, E, preferred_element_type=f32)

    out_ref[0] = (jax.lax.dot_general(attn_out.astype(bf16),
                                      wout_ref[...], _DNT,
                                      preferred_element_type=f32)
                  + bout_ref[...])


def kernel(x, Wqkv, bqkv, Wout, bout, group_scale, anchor_idx):
    B, S, D = x.shape
    dh = D // _H
    blk = 512
    nb = S // blk

    Wqkv_b = Wqkv.astype(jnp.bfloat16)
    Wout_b = Wout.astype(jnp.bfloat16)
    bq = bqkv[:D].reshape(1, D)
    bkv = bqkv[D:].reshape(1, 2 * D)
    bout2 = bout.reshape(1, D)
    lw = jnp.log(jax.nn.softmax(group_scale))[jnp.array(_GROUP)]

    grid = (B, nb)
    return pl.pallas_call(
        functools.partial(_fused_kernel, blk=blk, S=S, D=D, dh=dh, nb=nb),
        grid=grid,
        in_specs=[
            pl.BlockSpec((1, S, D), lambda b, i: (b, 0, 0)),
            pl.BlockSpec((3 * D, D), lambda b, i: (0, 0)),
            pl.BlockSpec((1, D), lambda b, i: (0, 0)),
            pl.BlockSpec((1, 2 * D), lambda b, i: (0, 0)),
            pl.BlockSpec((D, D), lambda b, i: (0, 0)),
            pl.BlockSpec((1, D), lambda b, i: (0, 0)),
            pl.BlockSpec(memory_space=pltpu.SMEM),
        ],
        out_specs=pl.BlockSpec((1, blk, D), lambda b, i: (b, i, 0)),
        out_shape=jax.ShapeDtypeStruct((B, S, D), jnp.float32),
        compiler_params=pltpu.CompilerParams(
            dimension_semantics=("parallel", "parallel")),
    )(x, Wqkv_b, bq, bkv, Wout_b, bout2, lw)


# speedup vs baseline: 1.5599x; 1.1027x over previous
import functools

import jax
import jax.numpy as jnp
from jax.experimental import pallas as pl
from jax.experimental.pallas import tpu as pltpu

_H = 12
_HALO = 10
_OFFS = (-3, -2, -1, 1, 2, 3, -10, -5, 5, 10)
_GROUP = (0, 0, 0, 0, 0, 0, 1, 1, 1, 1, 2, 2)

_DNT = (((1,), (1,)), ((), ()))


def _fused_kernel(x_ref, wqkv_ref, b_ref, wout_ref, bout_ref, gw_ref,
                  out_ref, wqkv_bf, wout_bf, *, blk, S, D, dh, nb):
    b = pl.program_id(0)
    i = pl.program_id(1)
    W = blk + 2 * _HALO
    f32 = jnp.float32
    bf16 = jnp.bfloat16

    @pl.when(jnp.logical_and(b == 0, i == 0))
    def _cast_weights():
        wqkv_bf[...] = wqkv_ref[...].astype(bf16)
        wout_bf[...] = wout_ref[...].astype(bf16)

    W2 = blk + 32
    wc = pl.multiple_of(jnp.clip(i * blk - 16, 0, S - W2), 16)
    xw = x_ref[0, pl.ds(wc, W2), :].astype(bf16)
    xh0 = jnp.concatenate(
        [jnp.broadcast_to(xw[0:1], (_HALO, D)), xw[:W - _HALO]], axis=0)
    xhN = jnp.concatenate(
        [xw[22:], jnp.broadcast_to(xw[W2 - 1:W2], (_HALO, D))], axis=0)
    xh = jnp.where(i == 0, xh0, jnp.where(i == nb - 1, xhN, xw[6:6 + W]))
    x0 = x_ref[0, pl.ds(0, 1), :].astype(bf16)
    xS = x_ref[0, pl.ds(S - 1, 1), :].astype(bf16)
    xkv = jnp.concatenate([xh, x0, xS], axis=0)

    q = (jax.lax.dot_general(xh[_HALO:_HALO + blk], wqkv_bf[0:D], _DNT,
                             preferred_element_type=f32)
         + b_ref[:, 0:D])
    kv = (jax.lax.dot_general(xkv, wqkv_bf[D:3 * D], _DNT,
                              preferred_element_type=f32)
          + b_ref[:, D:3 * D])
    kh, vh = kv[:, :D], kv[:, D:]

    scale = dh ** -0.5
    rows = jax.lax.broadcasted_iota(jnp.int32, (D, _H), 0)
    cols = jax.lax.broadcasted_iota(jnp.int32, (D, _H), 1)
    Ms = jnp.where(rows // dh == cols, scale, 0.0).astype(f32)
    rows_e = jax.lax.broadcasted_iota(jnp.int32, (_H, D), 0)
    cols_e = jax.lax.broadcasted_iota(jnp.int32, (_H, D), 1)
    E = jnp.where(cols_e // dh == rows_e, 1.0, 0.0).astype(f32)

    ks = [kh[_HALO + o:_HALO + o + blk] for o in _OFFS]
    ks += [kh[W:W + 1], kh[W + 1:W + 2]]
    vs = [vh[_HALO + o:_HALO + o + blk] for o in _OFFS]
    vs += [vh[W:W + 1], vh[W + 1:W + 2]]

    Ls = [jnp.dot(q * ks[a], Ms, preferred_element_type=f32)
          for a in range(12)]
    m = functools.reduce(jnp.maximum, Ls)
    acc = jnp.zeros((blk, D), f32)
    Z = jnp.zeros((blk, _H), f32)
    for a in range(12):
        e = jnp.exp(Ls[a] - m) * gw_ref[_GROUP[a]]
        Z = Z + e
        acc = acc + jnp.dot(e, E, preferred_element_type=f32) * vs[a]
    attn_out = acc / jnp.dot(Z, E, preferred_element_type=f32)

    out_ref[0] = (jax.lax.dot_general(attn_out.astype(bf16),
                                      wout_bf[...], _DNT,
                                      preferred_element_type=f32)
                  + bout_ref[...])


def kernel(x, Wqkv, bqkv, Wout, bout, group_scale, anchor_idx):
    B, S, D = x.shape
    dh = D // _H
    blk = 512
    nb = S // blk

    gw = jax.nn.softmax(group_scale)

    grid = (B, nb)
    return pl.pallas_call(
        functools.partial(_fused_kernel, blk=blk, S=S, D=D, dh=dh, nb=nb),
        grid=grid,
        in_specs=[
            pl.BlockSpec((1, S, D), lambda b, i: (b, 0, 0)),
            pl.BlockSpec((3 * D, D), lambda b, i: (0, 0)),
            pl.BlockSpec((1, 3 * D), lambda b, i: (0, 0)),
            pl.BlockSpec((D, D), lambda b, i: (0, 0)),
            pl.BlockSpec((1, D), lambda b, i: (0, 0)),
            pl.BlockSpec(memory_space=pltpu.SMEM),
        ],
        out_specs=pl.BlockSpec((1, blk, D), lambda b, i: (b, i, 0)),
        out_shape=jax.ShapeDtypeStruct((B, S, D), jnp.float32),
        scratch_shapes=[
            pltpu.VMEM((3 * D, D), jnp.bfloat16),
            pltpu.VMEM((D, D), jnp.bfloat16),
        ],
        compiler_params=pltpu.CompilerParams(
            dimension_semantics=("arbitrary", "arbitrary")),
    )(x, Wqkv, bqkv.reshape(1, 3 * D), Wout, bout.reshape(1, D), gw)


# speedup vs baseline: 1.5907x; 1.0197x over previous
import functools

import jax
import jax.numpy as jnp
from jax.experimental import pallas as pl
from jax.experimental.pallas import tpu as pltpu

_H = 12
_HALO = 10
_OFFS = (-3, -2, -1, 1, 2, 3, -10, -5, 5, 10)
_GROUP = (0, 0, 0, 0, 0, 0, 1, 1, 1, 1, 2, 2)

_DNT = (((1,), (1,)), ((), ()))


def _fused_kernel(x_ref, wqkv_ref, b_ref, wout_ref, bout_ref, gw_ref,
                  out_ref, wqkv_bf, wout_bf, *, blk, S, D, dh, nb):
    b = pl.program_id(0)
    i = pl.program_id(1)
    W = blk + 2 * _HALO
    f32 = jnp.float32
    bf16 = jnp.bfloat16

    @pl.when(jnp.logical_and(b == 0, i == 0))
    def _cast_weights():
        wqkv_bf[...] = wqkv_ref[...].astype(bf16)
        wout_bf[...] = wout_ref[...].astype(bf16)

    W2 = blk + 32
    wc = pl.multiple_of(jnp.clip(i * blk - 16, 0, S - W2), 16)
    xw = x_ref[0, pl.ds(wc, W2), :].astype(bf16)
    xh0 = jnp.concatenate(
        [jnp.broadcast_to(xw[0:1], (_HALO, D)), xw[:W - _HALO]], axis=0)
    xhN = jnp.concatenate(
        [xw[22:], jnp.broadcast_to(xw[W2 - 1:W2], (_HALO, D))], axis=0)
    xh = jnp.where(i == 0, xh0, jnp.where(i == nb - 1, xhN, xw[6:6 + W]))
    x0 = x_ref[0, pl.ds(0, 1), :].astype(bf16)
    xS = x_ref[0, pl.ds(S - 1, 1), :].astype(bf16)
    xkv = jnp.concatenate([xh, x0, xS], axis=0)

    q = (jax.lax.dot_general(xh[_HALO:_HALO + blk], wqkv_bf[0:D], _DNT,
                             preferred_element_type=f32)
         + b_ref[:, 0:D]).astype(bf16)
    kv = (jax.lax.dot_general(xkv, wqkv_bf[D:3 * D], _DNT,
                              preferred_element_type=f32)
          + b_ref[:, D:3 * D]).astype(bf16)
    kh, vh = kv[:, :D], kv[:, D:]

    scale = dh ** -0.5
    rows = jax.lax.broadcasted_iota(jnp.int32, (D, _H), 0)
    cols = jax.lax.broadcasted_iota(jnp.int32, (D, _H), 1)
    Ms = jnp.where(rows // dh == cols, scale, 0.0).astype(bf16)
    rows_e = jax.lax.broadcasted_iota(jnp.int32, (_H, D), 0)
    cols_e = jax.lax.broadcasted_iota(jnp.int32, (_H, D), 1)
    E = jnp.where(cols_e // dh == rows_e, 1.0, 0.0).astype(bf16)

    ks = [kh[_HALO + o:_HALO + o + blk] for o in _OFFS]
    ks += [kh[W:W + 1], kh[W + 1:W + 2]]
    vs = [vh[_HALO + o:_HALO + o + blk] for o in _OFFS]
    vs += [vh[W:W + 1], vh[W + 1:W + 2]]

    Ls = [jnp.dot(q * ks[a], Ms, preferred_element_type=f32)
          for a in range(12)]
    m = functools.reduce(jnp.maximum, Ls)
    acc = jnp.zeros((blk, D), f32)
    Z = jnp.zeros((blk, _H), f32)
    for a in range(12):
        e = jnp.exp(Ls[a] - m) * gw_ref[_GROUP[a]]
        Z = Z + e
        acc = acc + (jnp.dot(e.astype(bf16), E, preferred_element_type=f32)
                     * vs[a])
    attn_out = acc / jnp.dot(Z, E.astype(f32), preferred_element_type=f32)

    out_ref[0] = (jax.lax.dot_general(attn_out.astype(bf16),
                                      wout_bf[...], _DNT,
                                      preferred_element_type=f32)
                  + bout_ref[...])


def kernel(x, Wqkv, bqkv, Wout, bout, group_scale, anchor_idx):
    B, S, D = x.shape
    dh = D // _H
    blk = 512
    nb = S // blk

    gw = jax.nn.softmax(group_scale)

    grid = (B, nb)
    return pl.pallas_call(
        functools.partial(_fused_kernel, blk=blk, S=S, D=D, dh=dh, nb=nb),
        grid=grid,
        in_specs=[
            pl.BlockSpec((1, S, D), lambda b, i: (b, 0, 0)),
            pl.BlockSpec((3 * D, D), lambda b, i: (0, 0)),
            pl.BlockSpec((1, 3 * D), lambda b, i: (0, 0)),
            pl.BlockSpec((D, D), lambda b, i: (0, 0)),
            pl.BlockSpec((1, D), lambda b, i: (0, 0)),
            pl.BlockSpec(memory_space=pltpu.SMEM),
        ],
        out_specs=pl.BlockSpec((1, blk, D), lambda b, i: (b, i, 0)),
        out_shape=jax.ShapeDtypeStruct((B, S, D), jnp.float32),
        scratch_shapes=[
            pltpu.VMEM((3 * D, D), jnp.bfloat16),
            pltpu.VMEM((D, D), jnp.bfloat16),
        ],
        compiler_params=pltpu.CompilerParams(
            dimension_semantics=("arbitrary", "arbitrary")),
    )(x, Wqkv, bqkv.reshape(1, 3 * D), Wout, bout.reshape(1, D), gw)
